# baseline (speedup 1.0000x reference)
import functools

import jax
import jax.numpy as jnp
from jax import lax
from jax.experimental import pallas as pl
from jax.experimental.pallas import tpu as pltpu
from jax.experimental.pallas import tpu_sc as plsc

_VOCAB = 1000000
_D = 32
_B = 16384
_NC = 2
_NS = 16
_NW = _NC * _NS
_QPW = _B // _NW
_L = 16

_CW = 1024
_RANGE = 31744
_NCHUNK = 31
_W31_LO = 31 * _RANGE
_W31_SIZE = _VOCAB - _W31_LO
_HCAP = 2560
_CCAP = 256


def _phase1(src_hbm, dst_hbm, tableT_hbm, tail_hbm, vecs_hbm,
            bufa, bufb, tailbuf, idsbuf, hq, hx, cq, cx, stage,
            sema, semb, semi, sems):
    wid = lax.axis_index("s") * _NC + lax.axis_index("c")
    is31 = wid == _NW - 1
    lo = wid * _RANGE
    size_w = jnp.where(is31, _W31_SIZE, _RANGE)
    lanes = lax.iota(jnp.int32, _L)

    def fire(c, sem, buf):
        off = lo + c * _CW
        is_tail = jnp.logical_and(is31, c == 15)

        @pl.when(jnp.logical_or(jnp.logical_not(is31), c < 15))
        def _():
            pltpu.async_copy(tableT_hbm.at[:, pl.ds(off, _CW)], buf, sem)

        @pl.when(is_tail)
        def _():
            pltpu.async_copy(tableT_hbm.at[:, pl.ds(999424, 512)],
                             buf.at[:, pl.ds(0, 512)], sem)

    def wait(c, sem, buf):
        is_tail = jnp.logical_and(is31, c == 15)

        @pl.when(jnp.logical_or(jnp.logical_not(is31), c < 15))
        def _():
            pltpu.make_async_copy(tableT_hbm.at[:, pl.ds(0, _CW)], buf, sem).wait()

        @pl.when(is_tail)
        def _():
            pltpu.make_async_copy(tableT_hbm.at[:, pl.ds(0, 512)],
                                  buf.at[:, pl.ds(0, 512)], sem).wait()

    fire(0, sema, bufa)
    fire(1, semb, bufb)
    pltpu.async_copy(src_hbm, idsbuf.at[pl.ds(0, _B)], semi)
    pltpu.async_copy(dst_hbm, idsbuf.at[pl.ds(_B, _B)], semi)
    pltpu.async_copy(tail_hbm, tailbuf, semi)
    pltpu.make_async_copy(src_hbm, idsbuf.at[pl.ds(0, _B)], semi).wait()
    pltpu.make_async_copy(src_hbm, idsbuf.at[pl.ds(_B, _B)], semi).wait()
    pltpu.make_async_copy(tail_hbm, tailbuf, semi).wait()

    def scan(v, nh):
        ids = idsbuf[pl.ds(v * _L, _L)]
        x = ids - lo
        m = jnp.logical_and(ids >= lo, x < size_w)
        tag = v * _L + lanes
        plsc.store_compressed(hq.at[pl.ds(nh, _L)], tag, mask=m)
        plsc.store_compressed(hx.at[pl.ds(nh, _L)], x, mask=m)
        nh = nh + jnp.max(plsc.all_reduce_population_count(m))
        return jnp.minimum(nh, _HCAP - _L)

    nh = lax.fori_loop(0, 2 * _B // _L, scan, jnp.int32(0))

    def do_chunk(c, sem, buf):
        wait(c, sem, buf)
        clo = c * _CW
        is_tail_chunk = jnp.logical_and(is31, c == 15)
        def match(v, nc):
            xs = hx[pl.ds(v * _L, _L)]
            qs = hq[pl.ds(v * _L, _L)]
            xl = xs - clo
            valid = v * _L + lanes < nh
            m = jnp.logical_and(jnp.logical_and(xl >= 0, xl < _CW), valid)
            plsc.store_compressed(cq.at[pl.ds(nc, _L)], qs, mask=m)
            plsc.store_compressed(cx.at[pl.ds(nc, _L)], xl, mask=m)
            nc = nc + jnp.max(plsc.all_reduce_population_count(m))
            return jnp.minimum(nc, _CCAP - _L)

        nc = lax.fori_loop(0, (nh + _L - 1) // _L, match, jnp.int32(0))

        def gather_group(v, carry):
            xv = cx[pl.ds(v * _L, _L)]
            slots = v * _L + lanes
            xc = jnp.minimum(xv, _CW - 1)
            xt = jnp.clip(xv - 512, 0, 63)
            from_tail = jnp.logical_and(is_tail_chunk, xv >= 512)
            for dd in range(_D):
                ddv = jnp.full((_L,), dd, jnp.int32)
                vals = plsc.load_gather(buf, [ddv, xc])
                tvals = plsc.load_gather(tailbuf, [xt, ddv])
                vals = jnp.where(from_tail, tvals, vals)
                plsc.store_scatter(stage, [slots * _D + dd], vals)
            return carry

        lax.fori_loop(0, (nc + _L - 1) // _L, gather_group, 0)

        def drain_one():
            pltpu.make_async_copy(vecs_hbm.at[pl.ds(0, _D)],
                                  stage.at[pl.ds(0, _D)], sems).wait()

        def emit(h, carry):
            qv = cq[pl.ds((h // _L) * _L, _L)]
            e = jnp.max(jnp.where(lanes == (h - (h // _L) * _L), qv, 0))
            pltpu.async_copy(stage.at[pl.ds(h * _D, _D)],
                             vecs_hbm.at[pl.ds(e * _D, _D)], sems)
            @pl.when(h >= _L)
            def _():
                drain_one()
            return carry

        lax.fori_loop(0, nc, emit, 0)

        def drain(h, carry):
            drain_one()
            return carry

        lax.fori_loop(0, jnp.minimum(nc, _L), drain, 0)

    def chunk_step(c, carry):
        cur = lax.rem(c, 2)

        @pl.when(cur == 0)
        def _():
            do_chunk(c, sema, bufa)
            @pl.when(c + 2 < _NCHUNK)
            def _():
                fire(c + 2, sema, bufa)

        @pl.when(cur == 1)
        def _():
            do_chunk(c, semb, bufb)
            @pl.when(c + 2 < _NCHUNK)
            def _():
                fire(c + 2, semb, bufb)

        return carry

    lax.fori_loop(0, _NCHUNK, chunk_step, 0)


def _phase2(vecs_hbm, out_hbm, sbuf, dbuf, outv, sem):
    wid = lax.axis_index("s") * _NC + lax.axis_index("c")
    base = wid * _QPW * _D
    pltpu.async_copy(vecs_hbm.at[pl.ds(base, _QPW * _D)], sbuf, sem)
    pltpu.async_copy(vecs_hbm.at[pl.ds(_B * _D + base, _QPW * _D)], dbuf, sem)
    pltpu.make_async_copy(vecs_hbm.at[pl.ds(0, _QPW * _D)], sbuf, sem).wait()
    pltpu.make_async_copy(vecs_hbm.at[pl.ds(0, _QPW * _D)], dbuf, sem).wait()

    lanes = lax.iota(jnp.int32, _L)

    def body(g, carry):
        rows = (g * _L + lanes) * _D
        acc = jnp.zeros((_L,), jnp.float32)
        for dd in range(_D):
            sv = plsc.load_gather(sbuf, [rows + dd])
            dv = plsc.load_gather(dbuf, [rows + dd])
            acc = acc + sv * dv
        outv[pl.ds(g * _L, _L)] = 1.0 / (1.0 + jnp.exp(-acc))
        return carry

    lax.fori_loop(0, _QPW // _L, body, 0)
    pltpu.sync_copy(outv, out_hbm.at[pl.ds(wid * _QPW, _QPW)])


@jax.jit
def kernel(src_ids, dst_ids, table):
    mesh = plsc.VectorSubcoreMesh(core_axis_name="c", subcore_axis_name="s")
    k1 = functools.partial(
        pl.kernel,
        mesh=mesh,
        compiler_params=pltpu.CompilerParams(needs_layout_passes=False),
        out_type=jax.ShapeDtypeStruct((2 * _B * _D,), jnp.float32),
        scratch_types=[
            pltpu.VMEM((_D, _CW), jnp.float32),
            pltpu.VMEM((_D, _CW), jnp.float32),
            pltpu.VMEM((64, _D), jnp.float32),
            pltpu.VMEM((2 * _B,), jnp.int32),
            pltpu.VMEM((_HCAP,), jnp.int32),
            pltpu.VMEM((_HCAP,), jnp.int32),
            pltpu.VMEM((_CCAP,), jnp.int32),
            pltpu.VMEM((_CCAP,), jnp.int32),
            pltpu.VMEM((_CCAP * _D,), jnp.float32),
            pltpu.SemaphoreType.DMA,
            pltpu.SemaphoreType.DMA,
            pltpu.SemaphoreType.DMA,
            pltpu.SemaphoreType.DMA,
        ],
    )(_phase1)
    k2 = functools.partial(
        pl.kernel,
        mesh=mesh,
        compiler_params=pltpu.CompilerParams(needs_layout_passes=False),
        out_type=jax.ShapeDtypeStruct((_B,), jnp.float32),
        scratch_types=[
            pltpu.VMEM((_QPW * _D,), jnp.float32),
            pltpu.VMEM((_QPW * _D,), jnp.float32),
            pltpu.VMEM((_QPW,), jnp.float32),
            pltpu.SemaphoreType.DMA,
        ],
    )(_phase2)
    vecs = k1(src_ids, dst_ids, table.T, table[_VOCAB - 64:, :])
    return k2(vecs)
